# baseline (speedup 1.0000x reference)
import functools

import jax
import jax.numpy as jnp
from jax import lax
from jax.experimental import pallas as pl
from jax.experimental.pallas import tpu as pltpu
from jax.experimental.pallas import tpu_sc as plsc

VOCAB = 1000000
DIM = 64
BATCH = 4096
SEQ = 200

NUM_CORES = 2
NUM_SUBCORES = 16
NW = NUM_CORES * NUM_SUBCORES
ROWS_W = BATCH // NW
K = 4
NCH = ROWS_W // K

_mesh = plsc.VectorSubcoreMesh(core_axis_name="c", subcore_axis_name="s")


@functools.partial(
    pl.kernel,
    mesh=_mesh,
    out_type=jax.ShapeDtypeStruct((BATCH, SEQ, DIM), jnp.float32),
    compiler_params=pltpu.CompilerParams(use_tc_tiling_on_sc=False),
    scratch_types=[
        pltpu.VMEM((K, SEQ), jnp.int32),
        pltpu.VMEM((K, SEQ), jnp.int32),
        pltpu.VMEM((K, SEQ, DIM), jnp.float32),
        pltpu.VMEM((K, SEQ, DIM), jnp.float32),
        pltpu.SemaphoreType.DMA,
        pltpu.SemaphoreType.DMA,
        pltpu.SemaphoreType.DMA,
        pltpu.SemaphoreType.DMA,
        pltpu.SemaphoreType.DMA,
        pltpu.SemaphoreType.DMA,
    ],
)
def _gather_kernel(idx_hbm, table_hbm, out_hbm, idx0, idx1, rows0, rows1,
                   si0, si1, sg0, sg1, so0, so1):
    wid = lax.axis_index("s") * NUM_CORES + lax.axis_index("c")
    base = wid * ROWS_W

    idxb = (idx0, idx1)
    rows = (rows0, rows1)
    si = (si0, si1)
    sg = (sg0, sg1)
    so = (so0, so1)

    def start_idx(g, b):
        pltpu.async_copy(idx_hbm.at[pl.ds(base + g * K, K), :], idxb[b], si[b])

    def wait_idx(b):
        pltpu.make_async_copy(idx_hbm.at[pl.ds(base, K), :], idxb[b],
                              si[b]).wait()

    def start_gather(b):
        for jj in range(K):
            pltpu.async_copy(table_hbm.at[idxb[b].at[jj]], rows[b].at[jj],
                             sg[b])

    def wait_gather(b):
        for jj in range(K):
            pltpu.make_async_copy(table_hbm.at[idxb[b].at[jj]],
                                  rows[b].at[jj], sg[b]).wait()

    def start_out(g, b):
        pltpu.async_copy(rows[b], out_hbm.at[pl.ds(base + g * K, K)], so[b])

    def wait_out(b):
        pltpu.make_async_copy(rows[b], out_hbm.at[pl.ds(base, K)],
                              so[b]).wait()

    def step(g, b, first, prefetch):
        wait_idx(b)
        if not first:
            wait_out(b)
        start_gather(b)
        wait_gather(b)
        if prefetch:
            start_idx(g + 2, b)
        start_out(g, b)

    start_idx(0, 0)
    start_idx(1, 1)
    step(0, 0, True, True)
    step(1, 1, True, True)

    def body(j, carry):
        g0 = 2 * j + 2
        step(g0, 0, False, True)
        step(g0 + 1, 1, False, True)
        return carry

    lax.fori_loop(0, (NCH - 4) // 2, body, 0)

    step(NCH - 2, 0, False, False)
    step(NCH - 1, 1, False, False)
    wait_out(0)
    wait_out(1)


def kernel(input_ids, table):
    return _gather_kernel(input_ids.astype(jnp.int32), table)
